# Initial kernel scaffold; baseline (speedup 1.0000x reference)
import functools

import jax
import jax.numpy as jnp
import numpy as np
from jax import lax
from jax.experimental import pallas as pl
from jax.experimental.pallas import tpu as pltpu
from jax.experimental.pallas import tpu_sc as plsc

B = 16384
H = 50
D = 32
OUT_D = 3 * D
NC = 2
NS = 16
NW = NC * NS
RW = B // NW
C = 32
G = RW // C
CH = C * H
L = 16


def _sc_kernel(user_hbm, mem_hbm, utab_hbm, itab_hbm, rowid_hbm, out_hbm,
               idx_v, rows_v, accum_v, out_v, uidx_v, user_v, rowid_v, sem):
    wid = lax.axis_index("s") * NC + lax.axis_index("c")
    base = wid * RW

    zeros = jnp.zeros((L,), jnp.float32)
    inv_h = jnp.float32(1.0 / H)

    pltpu.sync_copy(rowid_hbm, rowid_v)

    pltpu.sync_copy(user_hbm.at[pl.ds(base, RW)], uidx_v)
    pltpu.async_copy(utab_hbm.at[uidx_v], user_v, sem).wait()

    def zero_body(r, _):
        accum_v[r, pl.ds(0, L)] = zeros
        accum_v[r, pl.ds(L, L)] = zeros
        return 0
    lax.fori_loop(0, C, zero_body, 0)

    def chunk_body(g, _):
        r0 = base + g * C
        pltpu.sync_copy(mem_hbm.at[pl.ds(pl.multiple_of(r0 * H, 8), CH)],
                        idx_v)
        pltpu.async_copy(itab_hbm.at[idx_v], rows_v, sem).wait()
        pltpu.sync_copy(rows_v, accum_v.at[rowid_v], add=True)

        def row_body(r, _):
            u_r = g * C + r
            for half in range(2):
                lo = half * L
                m = accum_v[r, pl.ds(lo, L)] * inv_h
                u = user_v[u_r, pl.ds(lo, L)]
                out_v[r, pl.ds(lo, L)] = m
                out_v[r, pl.ds(D + lo, L)] = m * u
                out_v[r, pl.ds(2 * D + lo, L)] = u
                accum_v[r, pl.ds(lo, L)] = zeros
            return 0
        lax.fori_loop(0, C, row_body, 0)

        pltpu.sync_copy(out_v, out_hbm.at[pl.ds(r0, C)])
        return 0

    lax.fori_loop(0, G, chunk_body, 0)


@jax.jit
def _run(user, mem_flat, user_table, item_table, row_ids):
    mesh = plsc.VectorSubcoreMesh(core_axis_name="c", subcore_axis_name="s")
    f = functools.partial(
        pl.kernel,
        mesh=mesh,
        out_type=jax.ShapeDtypeStruct((B, OUT_D), jnp.float32),
        scratch_types=[
            pltpu.VMEM((CH,), jnp.int32),
            pltpu.VMEM((CH, D), jnp.float32),
            pltpu.VMEM((C, D), jnp.float32),
            pltpu.VMEM((C, OUT_D), jnp.float32),
            pltpu.VMEM((RW,), jnp.int32),
            pltpu.VMEM((RW, D), jnp.float32),
            pltpu.VMEM((CH,), jnp.int32),
            pltpu.SemaphoreType.DMA,
        ],
    )(_sc_kernel)
    return f(user, mem_flat, user_table, item_table, row_ids)


_ROW_IDS = jnp.asarray(np.repeat(np.arange(C), H).astype(np.int32))


def kernel(user, memory, user_table, item_table):
    return _run(user, memory.reshape(-1), user_table, item_table, _ROW_IDS)


# speedup vs baseline: 2.5508x; 2.5508x over previous
import functools

import jax
import jax.numpy as jnp
import numpy as np
from jax import lax
from jax.experimental import pallas as pl
from jax.experimental.pallas import tpu as pltpu
from jax.experimental.pallas import tpu_sc as plsc

B = 16384
H = 50
D = 32
OUT_D = 3 * D
NC = 2
NS = 16
NW = NC * NS
RW = B // NW
C = 64
G = RW // C
CH = C * H
L = 16


def _sc_kernel(user_hbm, mem_hbm, utab_hbm, itab_hbm, rowid_hbm, out_hbm,
               idx_v, rows_v, accum_v, out_v, uidx_v, user_v, rowid_v,
               zeros_v, shacc, sem):
    sid = lax.axis_index("s")
    wid = sid * NC + lax.axis_index("c")
    base = wid * RW
    sbase = pl.multiple_of(sid * C, C)

    zeros = jnp.zeros((L,), jnp.float32)
    inv_h = jnp.float32(1.0 / H)

    pltpu.sync_copy(rowid_hbm, rowid_v)
    off = (sid * C).astype(jnp.int32)

    def off_body(i, _):
        rowid_v[pl.ds(i * L, L)] = rowid_v[pl.ds(i * L, L)] + off
        return 0
    lax.fori_loop(0, CH // L, off_body, 0)

    def zero_body(r, _):
        zeros_v[r, pl.ds(0, L)] = zeros
        zeros_v[r, pl.ds(L, L)] = zeros
        return 0
    lax.fori_loop(0, C, zero_body, 0)
    pltpu.sync_copy(zeros_v, shacc.at[pl.ds(sbase, C)])

    def chunk_body(g, _):
        r0 = base + g * C
        pltpu.sync_copy(mem_hbm.at[pl.ds(pl.multiple_of(r0 * H, 8), CH)],
                        idx_v)
        pltpu.async_copy(itab_hbm.at[idx_v], rows_v, sem).wait()
        pltpu.sync_copy(rows_v, shacc.at[rowid_v], add=True)
        pltpu.sync_copy(user_hbm.at[pl.ds(r0, C)], uidx_v)
        pltpu.async_copy(utab_hbm.at[uidx_v], user_v, sem).wait()
        pltpu.sync_copy(shacc.at[pl.ds(sbase, C)], accum_v)
        pltpu.sync_copy(zeros_v, shacc.at[pl.ds(sbase, C)])

        def row_body(r, _):
            for half in range(2):
                lo = half * L
                m = accum_v[r, pl.ds(lo, L)] * inv_h
                u = user_v[r, pl.ds(lo, L)]
                out_v[r, pl.ds(lo, L)] = m
                out_v[r, pl.ds(D + lo, L)] = m * u
                out_v[r, pl.ds(2 * D + lo, L)] = u
            return 0
        lax.fori_loop(0, C, row_body, 0)

        pltpu.sync_copy(out_v, out_hbm.at[pl.ds(r0, C)])
        return 0

    lax.fori_loop(0, G, chunk_body, 0)


@jax.jit
def _run(user, mem_flat, user_table, item_table, row_ids):
    mesh = plsc.VectorSubcoreMesh(core_axis_name="c", subcore_axis_name="s")
    f = functools.partial(
        pl.kernel,
        mesh=mesh,
        compiler_params=pltpu.CompilerParams(use_tc_tiling_on_sc=False),
        out_type=jax.ShapeDtypeStruct((B, OUT_D), jnp.float32),
        scratch_types=[
            pltpu.VMEM((CH,), jnp.int32),
            pltpu.VMEM((CH, D), jnp.float32),
            pltpu.VMEM((C, D), jnp.float32),
            pltpu.VMEM((C, OUT_D), jnp.float32),
            pltpu.VMEM((C,), jnp.int32),
            pltpu.VMEM((C, D), jnp.float32),
            pltpu.VMEM((CH,), jnp.int32),
            pltpu.VMEM((C, D), jnp.float32),
            pltpu.MemorySpace.VMEM_SHARED((NS * C, D), jnp.float32),
            pltpu.SemaphoreType.DMA,
        ],
    )(_sc_kernel)
    return f(user, mem_flat, user_table, item_table, row_ids)


_ROW_IDS = np.repeat(np.arange(C), H).astype(np.int32)


def kernel(user, memory, user_table, item_table):
    return _run(user, memory.reshape(-1), user_table, item_table, _ROW_IDS)
